# baseline (speedup 1.0000x reference)
import jax
import jax.numpy as jnp
from jax.experimental import pallas as pl

K = 2
IN_DIM = 2048
OUT_DIM = 4096
HID = 512
BLK_M = 512


def _fused_kernel(x_ref, g1_ref, b1_ref, w12_ref, w3_ref, g2_ref, b2_ref,
                  out_ref):
    x = x_ref[...]
    mean = jnp.mean(x, axis=-1, keepdims=True)
    xc = x - mean
    var = jnp.mean(xc * xc, axis=-1, keepdims=True)
    xn = xc * jax.lax.rsqrt(var + 1e-6) * g1_ref[...] + b1_ref[...]
    h = jax.lax.dot_general(xn.astype(jnp.bfloat16),
                            w12_ref[...].astype(jnp.bfloat16),
                            (((1,), (1,)), ((), ())),
                            preferred_element_type=jnp.float32)
    gate = h[:, :HID]
    val = h[:, HID:]
    act = gate * jax.nn.sigmoid(gate) * val
    y = jax.lax.dot_general(act.astype(jnp.bfloat16),
                            w3_ref[...].astype(jnp.bfloat16),
                            (((1,), (1,)), ((), ())),
                            preferred_element_type=jnp.float32)
    mean2 = jnp.mean(y, axis=-1, keepdims=True)
    yc = y - mean2
    var2 = jnp.mean(yc * yc, axis=-1, keepdims=True)
    out_ref[...] = yc * jax.lax.rsqrt(var2 + 1e-6) * g2_ref[...] + b2_ref[...]


def kernel(x, ln_pre_g, ln_pre_b, w12, w3, router_w, router_b, ln_post_g,
           ln_post_b):
    b, s, d = x.shape
    x_flat = x.reshape(-1, d * K)
    m = x_flat.shape[0]
    out = pl.pallas_call(
        _fused_kernel,
        grid=(m // BLK_M,),
        in_specs=[
            pl.BlockSpec((BLK_M, IN_DIM), lambda i: (i, 0)),
            pl.BlockSpec((1, IN_DIM), lambda i: (0, 0)),
            pl.BlockSpec((1, IN_DIM), lambda i: (0, 0)),
            pl.BlockSpec((2 * HID, IN_DIM), lambda i: (0, 0)),
            pl.BlockSpec((OUT_DIM, HID), lambda i: (0, 0)),
            pl.BlockSpec((1, OUT_DIM), lambda i: (0, 0)),
            pl.BlockSpec((1, OUT_DIM), lambda i: (0, 0)),
        ],
        out_specs=pl.BlockSpec((BLK_M, OUT_DIM), lambda i: (i, 0)),
        out_shape=jax.ShapeDtypeStruct((m, OUT_DIM), jnp.float32),
    )(x_flat, ln_pre_g.reshape(1, -1), ln_pre_b.reshape(1, -1), w12, w3,
      ln_post_g.reshape(1, -1), ln_post_b.reshape(1, -1))
    final = out.reshape(b, s // K, OUT_DIM)
    aux_loss = jnp.zeros((), jnp.float32)
    return (final, aux_loss)


# speedup vs baseline: 1.1817x; 1.1817x over previous
import jax
import jax.numpy as jnp
from jax.experimental import pallas as pl

K = 2
ENC = 1024
IN_DIM = 2048
OUT_DIM = 4096
HID = 512
BLK_M = 512
SEQ_BLK = K * BLK_M


def _fused_kernel(x_ref, g1_ref, b1_ref, w12_ref, w3_ref, g2_ref, b2_ref,
                  out_ref):
    x = x_ref[...].reshape(BLK_M, IN_DIM)
    mean = jnp.mean(x, axis=-1, keepdims=True)
    xc = x - mean
    var = jnp.mean(xc * xc, axis=-1, keepdims=True)
    xn = xc * jax.lax.rsqrt(var + 1e-6) * g1_ref[...] + b1_ref[...]
    h = jax.lax.dot_general(xn, w12_ref[...], (((1,), (1,)), ((), ())),
                            preferred_element_type=jnp.float32)
    gate = h[:, :HID]
    val = h[:, HID:]
    act = gate * jax.nn.sigmoid(gate) * val
    y = jax.lax.dot_general(act, w3_ref[...], (((1,), (1,)), ((), ())),
                            preferred_element_type=jnp.float32)
    mean2 = jnp.mean(y, axis=-1, keepdims=True)
    yc = y - mean2
    var2 = jnp.mean(yc * yc, axis=-1, keepdims=True)
    out_ref[...] = (yc * jax.lax.rsqrt(var2 + 1e-6) * g2_ref[...]
                    + b2_ref[...]).reshape(1, BLK_M, OUT_DIM)


def kernel(x, ln_pre_g, ln_pre_b, w12, w3, router_w, router_b, ln_post_g,
           ln_post_b):
    b, s, d = x.shape
    nb = s // SEQ_BLK
    out = pl.pallas_call(
        _fused_kernel,
        grid=(b, nb),
        in_specs=[
            pl.BlockSpec((1, SEQ_BLK, ENC), lambda i, j: (i, j, 0)),
            pl.BlockSpec((1, IN_DIM), lambda i, j: (0, 0)),
            pl.BlockSpec((1, IN_DIM), lambda i, j: (0, 0)),
            pl.BlockSpec((2 * HID, IN_DIM), lambda i, j: (0, 0)),
            pl.BlockSpec((OUT_DIM, HID), lambda i, j: (0, 0)),
            pl.BlockSpec((1, OUT_DIM), lambda i, j: (0, 0)),
            pl.BlockSpec((1, OUT_DIM), lambda i, j: (0, 0)),
        ],
        out_specs=pl.BlockSpec((1, BLK_M, OUT_DIM), lambda i, j: (i, j, 0)),
        out_shape=jax.ShapeDtypeStruct((b, s // K, OUT_DIM), jnp.float32),
    )(x, ln_pre_g.reshape(1, -1), ln_pre_b.reshape(1, -1), w12, w3,
      ln_post_g.reshape(1, -1), ln_post_b.reshape(1, -1))
    aux_loss = jnp.zeros((), jnp.float32)
    return (out, aux_loss)
